# baseline (speedup 1.0000x reference)
import jax
import jax.numpy as jnp
from jax.experimental import pallas as pl
from jax.experimental.pallas import tpu as pltpu

_TAPS = [(ky, kx) for ky in range(3) for kx in range(3)] + [(0, 0)]


def _fused_body(idx_ref, ps_ref, xpad_ref, w1_ref, b1_ref, w2_ref, b2_ref,
                aW1_ref, ab1_ref, aW2_ref, ab2_ref, out_ref, h1pad_ref):
    n = pl.program_id(0)
    _, H, W, C = out_ref.shape
    HW = H * W
    p1 = ps_ref[0]
    p2 = ps_ref[1]

    def shifted(src3d):
        return (src3d[:, 0:W, :], src3d[:, 1:W + 1, :], src3d[:, 2:W + 2, :])

    def conv(sh, w_ref):
        acc = None
        for p in range(5):
            (ky_a, kx_a), (ky_b, kx_b) = _TAPS[2 * p], _TAPS[2 * p + 1]
            lhs = jnp.concatenate(
                [sh[kx_a][ky_a:ky_a + H].reshape(HW, C),
                 sh[kx_b][ky_b:ky_b + H].reshape(HW, C)], axis=1)
            d = jnp.dot(lhs, w_ref[p], preferred_element_type=jnp.float32)
            acc = d if acc is None else acc + d
        return acc

    xsh = shifted(xpad_ref[0])
    h1 = conv(xsh, w1_ref) + b1_ref[...]
    h1 = jnp.where(h1 >= 0, h1, p1 * h1)

    zrow = jnp.zeros((1, W + 2, C), jnp.bfloat16)
    h1pad_ref[0:1] = zrow
    h1pad_ref[H + 1:H + 2] = zrow
    zcol = jnp.zeros((H + 2, 1, C), jnp.bfloat16)
    h1pad_ref[:, 0:1, :] = zcol
    h1pad_ref[:, W + 1:W + 2, :] = zcol
    h1pad_ref[1:H + 1, 1:W + 1, :] = h1.reshape(H, W, C).astype(jnp.bfloat16)

    hsh = shifted(h1pad_ref[...])
    h2 = conv(hsh, w2_ref) + b2_ref[...]

    x1 = jnp.sum(h2, axis=0, keepdims=True) * (1.0 / HW)
    e = idx_ref[n]
    a = jnp.dot(x1, aW1_ref[e], preferred_element_type=jnp.float32)
    a = jnp.maximum(a + ab1_ref[e], 0.0)
    g = jnp.dot(a, aW2_ref[e], preferred_element_type=jnp.float32)
    g = g + ab2_ref[e]
    s = jax.nn.sigmoid(g)

    xin = xsh[1][1:H + 1].reshape(HW, C).astype(jnp.float32)
    o = h2 * s + xin
    o = jnp.where(o >= 0, o, p2 * o)
    out_ref[0] = o.reshape(H, W, C)


def kernel(x, intensity, conv1_w, conv1_b, prelu1, conv2_w, conv2_b,
           aW1, ab1, aW2, ab2, prelu2):
    N, C, H, W = x.shape
    CH = aW1.shape[1]

    xh = jnp.transpose(x, (0, 2, 3, 1))
    xpad = jnp.pad(xh, ((0, 0), (1, 1), (1, 1), (0, 0))).astype(jnp.bfloat16)

    def prep_w(w):
        wt = jnp.transpose(w, (2, 3, 1, 0)).reshape(9, C, C)
        wt = jnp.concatenate([wt, jnp.zeros((1, C, C), wt.dtype)], axis=0)
        return wt.reshape(5, 2 * C, C).astype(jnp.bfloat16)

    w1p = prep_w(conv1_w)
    w2p = prep_w(conv2_w)
    b1 = conv1_b.reshape(1, C)
    b2 = conv2_b.reshape(1, C)
    aW1t = jnp.transpose(aW1, (0, 2, 1))
    aW2t = jnp.transpose(aW2, (0, 2, 1))
    ab1r = ab1.reshape(3, 1, CH)
    ab2r = ab2.reshape(3, 1, C)
    idx = (intensity - 1).astype(jnp.int32)
    ps = jnp.stack([prelu1, prelu2]).astype(jnp.float32)

    grid_spec = pltpu.PrefetchScalarGridSpec(
        num_scalar_prefetch=2,
        grid=(N,),
        in_specs=[
            pl.BlockSpec((1, H + 2, W + 2, C), lambda n, *_: (n, 0, 0, 0)),
            pl.BlockSpec((5, 2 * C, C), lambda n, *_: (0, 0, 0)),
            pl.BlockSpec((1, C), lambda n, *_: (0, 0)),
            pl.BlockSpec((5, 2 * C, C), lambda n, *_: (0, 0, 0)),
            pl.BlockSpec((1, C), lambda n, *_: (0, 0)),
            pl.BlockSpec((3, C, CH), lambda n, *_: (0, 0, 0)),
            pl.BlockSpec((3, 1, CH), lambda n, *_: (0, 0, 0)),
            pl.BlockSpec((3, CH, C), lambda n, *_: (0, 0, 0)),
            pl.BlockSpec((3, 1, C), lambda n, *_: (0, 0, 0)),
        ],
        out_specs=pl.BlockSpec((1, H, W, C), lambda n, *_: (n, 0, 0, 0)),
        scratch_shapes=[pltpu.VMEM((H + 2, W + 2, C), jnp.bfloat16)],
    )
    out = pl.pallas_call(
        _fused_body,
        out_shape=jax.ShapeDtypeStruct((N, H, W, C), jnp.float32),
        grid_spec=grid_spec,
        compiler_params=pltpu.CompilerParams(
            dimension_semantics=("parallel",),
            vmem_limit_bytes=60 * 1024 * 1024,
        ),
        name="fused_domain_adaption",
    )(idx, ps, xpad, w1p, b1, w2p, b2, aW1t, ab1r, aW2t, ab2r)
    return jnp.transpose(out, (0, 3, 1, 2))


# speedup vs baseline: 1.0520x; 1.0520x over previous
import jax
import jax.numpy as jnp
from jax.experimental import pallas as pl
from jax.experimental.pallas import tpu as pltpu

_TAPS = [(ky, kx) for ky in range(3) for kx in range(3)] + [(0, 0)]
_RC = 32


def _fused_body(idx_ref, ps_ref, x_ref, w1_ref, b1_ref, w2_ref, b2_ref,
                aW1_ref, ab1_ref, aW2_ref, ab2_ref, out_ref):
    n = pl.program_id(0)
    _, H, W, C = x_ref.shape
    HW = H * W
    p1 = ps_ref[0]
    p2 = ps_ref[1]

    zrow = jnp.zeros((1, W, C), jnp.float32)
    zcol = jnp.zeros((H, 1, C), jnp.float32)

    def shifted_copies(src):
        left = jnp.concatenate([zcol, src[:, 0:W - 1, :]], axis=1)
        right = jnp.concatenate([src[:, 1:W, :], zcol], axis=1)

        def hpad(v):
            return jnp.concatenate([zrow, v, zrow], axis=0).astype(jnp.bfloat16)

        return (hpad(left), hpad(src), hpad(right))

    def conv(sh, w_ref):
        outs = []
        for r0 in range(0, H, _RC):
            acc = None
            for p in range(5):
                (ky_a, kx_a), (ky_b, kx_b) = _TAPS[2 * p], _TAPS[2 * p + 1]
                lhs = jnp.concatenate(
                    [sh[kx_a][ky_a + r0:ky_a + r0 + _RC].reshape(_RC * W, C),
                     sh[kx_b][ky_b + r0:ky_b + r0 + _RC].reshape(_RC * W, C)],
                    axis=1)
                d = jnp.dot(lhs, w_ref[p], preferred_element_type=jnp.float32)
                acc = d if acc is None else acc + d
            outs.append(acc)
        return jnp.concatenate(outs, axis=0)

    xs = shifted_copies(x_ref[0])
    h1 = conv(xs, w1_ref) + b1_ref[...]
    h1 = jnp.where(h1 >= 0, h1, p1 * h1)

    hs = shifted_copies(h1.reshape(H, W, C))
    h2 = conv(hs, w2_ref) + b2_ref[...]

    x1 = jnp.sum(h2, axis=0, keepdims=True) * (1.0 / HW)
    e = idx_ref[n]
    a = jnp.dot(x1, aW1_ref[e], preferred_element_type=jnp.float32)
    a = jnp.maximum(a + ab1_ref[e], 0.0)
    g = jnp.dot(a, aW2_ref[e], preferred_element_type=jnp.float32)
    g = g + ab2_ref[e]
    s = jax.nn.sigmoid(g)

    xin = xs[1][1:H + 1].reshape(HW, C).astype(jnp.float32)
    o = h2 * s + xin
    o = jnp.where(o >= 0, o, p2 * o)
    out_ref[0] = o.reshape(H, W, C).astype(jnp.bfloat16)


def kernel(x, intensity, conv1_w, conv1_b, prelu1, conv2_w, conv2_b,
           aW1, ab1, aW2, ab2, prelu2):
    N, C, H, W = x.shape
    CH = aW1.shape[1]

    xh = jnp.transpose(x, (0, 2, 3, 1))

    def prep_w(w):
        wt = jnp.transpose(w, (2, 3, 1, 0)).reshape(9, C, C)
        wt = jnp.concatenate([wt, jnp.zeros((1, C, C), wt.dtype)], axis=0)
        return wt.reshape(5, 2 * C, C).astype(jnp.bfloat16)

    w1p = prep_w(conv1_w)
    w2p = prep_w(conv2_w)
    b1 = conv1_b.reshape(1, C)
    b2 = conv2_b.reshape(1, C)
    aW1t = jnp.transpose(aW1, (0, 2, 1))
    aW2t = jnp.transpose(aW2, (0, 2, 1))
    ab1r = ab1.reshape(3, 1, CH)
    ab2r = ab2.reshape(3, 1, C)
    idx = (intensity - 1).astype(jnp.int32)
    ps = jnp.stack([prelu1, prelu2]).astype(jnp.float32)

    grid_spec = pltpu.PrefetchScalarGridSpec(
        num_scalar_prefetch=2,
        grid=(N,),
        in_specs=[
            pl.BlockSpec((1, H, W, C), lambda n, *_: (n, 0, 0, 0)),
            pl.BlockSpec((5, 2 * C, C), lambda n, *_: (0, 0, 0)),
            pl.BlockSpec((1, C), lambda n, *_: (0, 0)),
            pl.BlockSpec((5, 2 * C, C), lambda n, *_: (0, 0, 0)),
            pl.BlockSpec((1, C), lambda n, *_: (0, 0)),
            pl.BlockSpec((3, C, CH), lambda n, *_: (0, 0, 0)),
            pl.BlockSpec((3, 1, CH), lambda n, *_: (0, 0, 0)),
            pl.BlockSpec((3, CH, C), lambda n, *_: (0, 0, 0)),
            pl.BlockSpec((3, 1, C), lambda n, *_: (0, 0, 0)),
        ],
        out_specs=pl.BlockSpec((1, H, W, C), lambda n, *_: (n, 0, 0, 0)),
    )
    out = pl.pallas_call(
        _fused_body,
        out_shape=jax.ShapeDtypeStruct((N, H, W, C), jnp.bfloat16),
        grid_spec=grid_spec,
        compiler_params=pltpu.CompilerParams(
            dimension_semantics=("parallel",),
            vmem_limit_bytes=60 * 1024 * 1024,
        ),
        name="fused_domain_adaption",
    )(idx, ps, xh, w1p, b1, w2p, b2, aW1t, ab1r, aW2t, ab2r)
    return jnp.transpose(out, (0, 3, 1, 2)).astype(jnp.float32)
